# baseline (speedup 1.0000x reference)
import jax
import jax.numpy as jnp
from jax import lax
from jax.experimental import pallas as pl
from jax.experimental.pallas import tpu as pltpu
from jax.experimental.pallas import tpu_sc as plsc

N_ATOMS = 500000
N_TYPES = 100
N_PROPS = 128
N_SYSTEMS = 2048

NC = 2
NS = 16
LANES = 16

HALF = N_ATOMS // NC
NGRP = 32
GSYS = N_SYSTEMS // NGRP
GBINS = GSYS * 128
NQ = NGRP // NS

SA = 15616
CBA = HALF - (NS - 1) * SA
NVA = CBA // LANES

CSZ = 16384
CMAXS = N_ATOMS - CSZ

assert CBA >= SA and CBA % LANES == 0 and SA % 8 == 0
assert CMAXS % 8 == 0


def _sc_hist_body(types_hbm, sys_hbm, out_hbm,
                  sys_v, types_v, hist_v, grploc_v, idx32_v, gbuf_v, shared_g):
    c = lax.axis_index("c")
    s = lax.axis_index("s")
    half_lo = c * HALF
    iota16 = lax.iota(jnp.int32, LANES)

    def zero_hist(i):
        hist_v[i // 8, pl.ds((i % 8) * LANES, LANES)] = (
            jnp.zeros((LANES,), jnp.float32))
    plsc.parallel_loop(0, NQ * GBINS // LANES, unroll=8)(zero_hist)
    for v in range(NGRP // LANES):
        grploc_v[pl.ds(v * LANES, LANES)] = jnp.zeros((LANES,), jnp.float32)
        idx32_v[pl.ds(v * LANES, LANES)] = iota16 + v * LANES
        gbuf_v[pl.ds(v * LANES, LANES)] = jnp.zeros((LANES,), jnp.float32)

    @pl.when(s == 0)
    def _zero_shared():
        pltpu.sync_copy(gbuf_v, shared_g)

    with jax.named_scope("ph_A"):
        baseA = half_lo + s * SA
        limitA = jnp.where(s == NS - 1, CBA, SA)
        pltpu.sync_copy(sys_hbm.at[pl.ds(baseA, CBA)], sys_v.at[pl.ds(0, CBA)])

        def count_body(i):
            sy = sys_v[pl.ds(i * LANES, LANES)]
            grp = lax.shift_right_logical(sy, 6)
            el = (i * LANES + iota16) < limitA
            cnt, last = plsc.scan_count(grp, mask=el)
            plsc.addupdate_scatter(grploc_v, [grp], cnt.astype(jnp.float32),
                                   mask=last)
        plsc.parallel_loop(0, NVA, unroll=3)(count_body)

    with jax.named_scope("ph_merge"):
        plsc.subcore_barrier()
        pltpu.sync_copy(grploc_v, shared_g.at[idx32_v], add=True)
        plsc.subcore_barrier()
        pltpu.sync_copy(shared_g, gbuf_v)

    with jax.named_scope("ph_C"):
        for q in range(NQ):
            gq = s + q * NS
            start_i = jnp.int32(0)
            n_i = jnp.int32(0)
            for v in range(NGRP // LANES):
                cv = gbuf_v[pl.ds(v * LANES, LANES)].astype(jnp.int32)
                jv = iota16 + v * LANES
                start_i += jnp.sum(jnp.where(jv < gq, cv, 0))
                n_i += jnp.sum(jnp.where(jv == gq, cv, 0))
            start_abs = half_lo + start_i
            nq_i = n_i
            k_first = lax.shift_right_logical(start_abs, 14)
            k_last = lax.shift_right_logical(start_abs + nq_i - 1, 14)
            trip = jnp.where(nq_i > 0, k_last - k_first + 1, 0)
            qoff = q * GBINS
            sys0 = gq * GSYS

            def chunk_body(ck, _, *, k_first=k_first, start_abs=start_abs,
                           nq_i=nq_i, qoff=qoff, sys0=sys0):
                k = k_first + ck
                cstart = jnp.minimum(k * CSZ, CMAXS)
                pltpu.sync_copy(types_hbm.at[pl.ds(cstart, CSZ)], types_v)
                pltpu.sync_copy(sys_hbm.at[pl.ds(cstart, CSZ)], sys_v)
                lo = jnp.maximum(k * CSZ, start_abs)
                hi = jnp.minimum((k + 1) * CSZ, start_abs + nq_i)
                i_lo = lax.shift_right_logical(lo - cstart, 4)
                i_hi = lax.shift_right_logical(hi - cstart + 15, 4)

                def vec_body(i):
                    sy = sys_v[pl.ds(i * LANES, LANES)]
                    t = types_v[pl.ds(i * LANES, LANES)]
                    comb = (sy - sys0) * 128 + t + qoff
                    posv = cstart + i * LANES + iota16
                    m = (posv >= lo) & (posv < hi)
                    cnt, last = plsc.scan_count(comb, mask=m)
                    plsc.addupdate_scatter(
                        hist_v, [lax.shift_right_logical(comb, 7), comb & 127],
                        cnt.astype(jnp.float32), mask=last)
                plsc.parallel_loop(i_lo, i_hi, unroll=3)(vec_body)
                return _

            lax.fori_loop(0, trip, chunk_body, None)

    with jax.named_scope("ph_out"):
        for q in range(NQ):
            gq = s + q * NS
            pltpu.sync_copy(hist_v.at[pl.ds(q * GSYS, GSYS)],
                            out_hbm.at[c, pl.ds(gq * GSYS, GSYS)])


def _sc_hist(atom_types, system_indices):
    mesh = plsc.VectorSubcoreMesh(core_axis_name="c", subcore_axis_name="s")
    return pl.kernel(
        _sc_hist_body,
        out_type=jax.ShapeDtypeStruct((NC, N_SYSTEMS, 128), jnp.float32),
        mesh=mesh,
        compiler_params=pltpu.CompilerParams(needs_layout_passes=False),
        scratch_types=[
            pltpu.VMEM((CSZ,), jnp.int32),
            pltpu.VMEM((CSZ,), jnp.int32),
            pltpu.VMEM((NQ * GSYS, 128), jnp.float32),
            pltpu.VMEM((NGRP,), jnp.float32),
            pltpu.VMEM((NGRP,), jnp.int32),
            pltpu.VMEM((NGRP,), jnp.float32),
            pltpu.VMEM_SHARED((NGRP,), jnp.float32),
        ],
    )(atom_types, system_indices)


def _tc_matmul_body(hist_ref, w_ref, t2i_ref, out_ref):
    h = hist_ref[0] + hist_ref[1]
    r = lax.broadcasted_iota(jnp.int32, (128, 128), 1)
    m = (t2i_ref[...] == r).astype(jnp.float32)
    w_eff = jnp.dot(m, w_ref[...], preferred_element_type=jnp.float32)
    out_ref[...] = jnp.dot(h, w_eff, preferred_element_type=jnp.float32)


def _tc_matmul(hist, w_pad, t2i_pad):
    blk = 256
    return pl.pallas_call(
        _tc_matmul_body,
        grid=(N_SYSTEMS // blk,),
        out_shape=jax.ShapeDtypeStruct((N_SYSTEMS, N_PROPS), jnp.float32),
        in_specs=[
            pl.BlockSpec((NC, blk, 128), lambda i: (0, i, 0)),
            pl.BlockSpec((128, 128), lambda i: (0, 0)),
            pl.BlockSpec((128, 1), lambda i: (0, 0)),
        ],
        out_specs=pl.BlockSpec((blk, N_PROPS), lambda i: (i, 0)),
    )(hist, w_pad, t2i_pad)


def kernel(atom_types, system_indices, weights, type_to_index):
    hist = _sc_hist(atom_types, system_indices)
    w_pad = jnp.pad(weights, ((0, 128 - N_TYPES), (0, 0)))
    t2i_pad = jnp.pad(type_to_index, (0, 128 - N_TYPES),
                      constant_values=127).reshape(128, 1)
    return _tc_matmul(hist, w_pad, t2i_pad)


# speedup vs baseline: 1.1809x; 1.1809x over previous
import jax
import jax.numpy as jnp
from jax import lax
from jax.experimental import pallas as pl
from jax.experimental.pallas import tpu as pltpu
from jax.experimental.pallas import tpu_sc as plsc

N_ATOMS = 500000
N_TYPES = 100
N_PROPS = 128
N_SYSTEMS = 2048

NC = 2
NS = 16
LANES = 16

HALF = N_ATOMS // NC
NGRP = 32
GSYS = N_SYSTEMS // NGRP
GBINS = GSYS * 128
NQ = NGRP // NS

SA = 15616
CBA = HALF - (NS - 1) * SA
NVA = CBA // LANES

CSZ = 16384
CMAXS = N_ATOMS - CSZ

assert CBA >= SA and CBA % LANES == 0 and SA % 8 == 0
assert CMAXS % 8 == 0


def _sc_hist_body(types_hbm, sys_hbm, out_hbm,
                  sys_v, types_v, hist_v, grploc_v, idx32_v, gbuf_v, shared_g,
                  sem_t, sem_s):
    c = lax.axis_index("c")
    s = lax.axis_index("s")
    half_lo = c * HALF
    iota16 = lax.iota(jnp.int32, LANES)

    def zero_hist(i):
        hist_v[i // 8, pl.ds((i % 8) * LANES, LANES)] = (
            jnp.zeros((LANES,), jnp.float32))
    plsc.parallel_loop(0, NQ * GBINS // LANES, unroll=8)(zero_hist)
    for v in range(NGRP // LANES):
        grploc_v[pl.ds(v * LANES, LANES)] = jnp.zeros((LANES,), jnp.float32)
        idx32_v[pl.ds(v * LANES, LANES)] = iota16 + v * LANES
        gbuf_v[pl.ds(v * LANES, LANES)] = jnp.zeros((LANES,), jnp.float32)

    @pl.when(s == 0)
    def _zero_shared():
        pltpu.sync_copy(gbuf_v, shared_g)

    with jax.named_scope("ph_A"):
        baseA = half_lo + s * SA
        limitA = jnp.where(s == NS - 1, CBA, SA)
        pltpu.sync_copy(sys_hbm.at[pl.ds(baseA, CBA)], sys_v.at[pl.ds(0, CBA)])

        def count_body(i):
            sy = sys_v[pl.ds(i * LANES, LANES)]
            grp = lax.shift_right_logical(sy, 6)
            el = (i * LANES + iota16) < limitA
            cnt, last = plsc.scan_count(grp, mask=el)
            plsc.addupdate_scatter(grploc_v, [grp], cnt.astype(jnp.float32),
                                   mask=last)
        plsc.parallel_loop(0, NVA, unroll=3)(count_body)

    with jax.named_scope("ph_merge"):
        plsc.subcore_barrier()
        pltpu.sync_copy(grploc_v, shared_g.at[idx32_v], add=True)
        plsc.subcore_barrier()
        pltpu.sync_copy(shared_g, gbuf_v)

    with jax.named_scope("ph_C"):
        for q in range(NQ):
            gq = s + q * NS
            start_i = jnp.int32(0)
            n_i = jnp.int32(0)
            for v in range(NGRP // LANES):
                cv = gbuf_v[pl.ds(v * LANES, LANES)].astype(jnp.int32)
                jv = iota16 + v * LANES
                start_i += jnp.sum(jnp.where(jv < gq, cv, 0))
                n_i += jnp.sum(jnp.where(jv == gq, cv, 0))
            start_abs = half_lo + start_i
            nq_i = n_i
            k_first = lax.shift_right_logical(start_abs, 14)
            k_last = lax.shift_right_logical(start_abs + nq_i - 1, 14)
            trip = jnp.where(nq_i > 0, k_last - k_first + 1, 0)
            qoff = q * GBINS
            sys0 = gq * GSYS

            def chunk_body(ck, _, *, k_first=k_first, start_abs=start_abs,
                           nq_i=nq_i, qoff=qoff, sys0=sys0):
                k = k_first + ck
                cstart = jnp.minimum(k * CSZ, CMAXS)
                cp_t = pltpu.async_copy(
                    types_hbm.at[pl.ds(cstart, CSZ)], types_v, sem_t)
                cp_s = pltpu.async_copy(
                    sys_hbm.at[pl.ds(cstart, CSZ)], sys_v, sem_s)
                cp_t.wait()
                cp_s.wait()
                lo = jnp.maximum(k * CSZ, start_abs)
                hi = jnp.minimum((k + 1) * CSZ, start_abs + nq_i)
                i_lo = lax.shift_right_logical(lo - cstart, 4)
                i_hi = lax.shift_right_logical(hi - cstart + 15, 4)

                def vec_body(i):
                    sy = sys_v[pl.ds(i * LANES, LANES)]
                    t = types_v[pl.ds(i * LANES, LANES)]
                    comb = (sy - sys0) * 128 + t + qoff
                    posv = cstart + i * LANES + iota16
                    m = (posv >= lo) & (posv < hi)
                    cnt, last = plsc.scan_count(comb, mask=m)
                    plsc.addupdate_scatter(
                        hist_v, [lax.shift_right_logical(comb, 7), comb & 127],
                        cnt.astype(jnp.float32), mask=last)
                plsc.parallel_loop(i_lo, i_hi, unroll=3)(vec_body)
                return _

            lax.fori_loop(0, trip, chunk_body, None)

    with jax.named_scope("ph_out"):
        for q in range(NQ):
            gq = s + q * NS
            pltpu.sync_copy(hist_v.at[pl.ds(q * GSYS, GSYS)],
                            out_hbm.at[c, pl.ds(gq * GSYS, GSYS)])


def _sc_hist(atom_types, system_indices):
    mesh = plsc.VectorSubcoreMesh(core_axis_name="c", subcore_axis_name="s")
    return pl.kernel(
        _sc_hist_body,
        out_type=jax.ShapeDtypeStruct((NC, N_SYSTEMS, 128), jnp.float32),
        mesh=mesh,
        compiler_params=pltpu.CompilerParams(needs_layout_passes=False),
        scratch_types=[
            pltpu.VMEM((CSZ,), jnp.int32),
            pltpu.VMEM((CSZ,), jnp.int32),
            pltpu.VMEM((NQ * GSYS, 128), jnp.float32),
            pltpu.VMEM((NGRP,), jnp.float32),
            pltpu.VMEM((NGRP,), jnp.int32),
            pltpu.VMEM((NGRP,), jnp.float32),
            pltpu.VMEM_SHARED((NGRP,), jnp.float32),
            pltpu.SemaphoreType.DMA,
            pltpu.SemaphoreType.DMA,
        ],
    )(atom_types, system_indices)


def _tc_matmul_body(hist_ref, w_ref, t2i_ref, out_ref):
    h = hist_ref[0] + hist_ref[1]
    r = lax.broadcasted_iota(jnp.int32, (128, 128), 1)
    m = (t2i_ref[...] == r).astype(jnp.float32)
    w_eff = jnp.dot(m, w_ref[...], preferred_element_type=jnp.float32)
    out_ref[...] = jnp.dot(h, w_eff, preferred_element_type=jnp.float32)


def _tc_matmul(hist, w_pad, t2i_pad):
    return pl.pallas_call(
        _tc_matmul_body,
        out_shape=jax.ShapeDtypeStruct((N_SYSTEMS, N_PROPS), jnp.float32),
        in_specs=[
            pl.BlockSpec(memory_space=pltpu.VMEM),
            pl.BlockSpec(memory_space=pltpu.VMEM),
            pl.BlockSpec(memory_space=pltpu.VMEM),
        ],
        out_specs=pl.BlockSpec(memory_space=pltpu.VMEM),
    )(hist, w_pad, t2i_pad)


def kernel(atom_types, system_indices, weights, type_to_index):
    hist = _sc_hist(atom_types, system_indices)
    w_pad = jnp.pad(weights, ((0, 128 - N_TYPES), (0, 0)))
    t2i_pad = jnp.pad(type_to_index, (0, 128 - N_TYPES),
                      constant_values=127).reshape(128, 1)
    return _tc_matmul(hist, w_pad, t2i_pad)
